# Initial kernel scaffold; baseline (speedup 1.0000x reference)
import jax
import jax.numpy as jnp
from jax.experimental import pallas as pl


def kernel(x, edge_index, W1, b1, W2, b2):
    raise NotImplementedError("write your pallas kernel here")


# speedup vs baseline: 11.6300x; 11.6300x over previous
import functools

import jax
import jax.numpy as jnp
from jax import lax
from jax.experimental import pallas as pl
from jax.experimental.pallas import tpu as pltpu
from jax.experimental.pallas import tpu_sc as plsc

N = 10000
NPAD = 10240
DIN = 128
D = 64
DT = 128
E = 320000
LANES = 128
CH = 79
NC = 2
NS = 16
NW = NC * NS
EPAD = NW * CH * LANES
RPS = NPAD // NS
BLK = 512
GRID = NPAD // BLK

_mesh = plsc.VectorSubcoreMesh(core_axis_name="c", subcore_axis_name="s")


def _deg_body(dst_hbm, out_hbm, dst_idx, ones_v, zbuf, acc):
    cid = lax.axis_index("c")
    sid = lax.axis_index("s")
    wid = sid * NC + cid

    def fill(i, carry):
        zbuf[pl.ds(i * 16, 16)] = jnp.zeros((16,), jnp.float32)
        return carry

    lax.fori_loop(0, RPS // 16, fill, 0)
    for k in range(LANES // 16):
        ones_v[pl.ds(k * 16, 16)] = jnp.ones((16,), jnp.float32)

    pltpu.sync_copy(zbuf, acc.at[pl.ds(sid * RPS, RPS)])
    plsc.subcore_barrier()

    pltpu.sync_copy(dst_hbm.at[wid], dst_idx)

    def step(j, carry):
        pltpu.sync_copy(ones_v, acc.at[dst_idx.at[j]], add=True)
        return carry

    lax.fori_loop(0, CH, step, 0)
    plsc.subcore_barrier()

    pltpu.sync_copy(acc.at[pl.ds(sid * RPS, RPS)], zbuf)
    pltpu.sync_copy(zbuf, out_hbm.at[cid, pl.ds(sid * RPS, RPS)])


_deg_kernel = pl.kernel(
    _deg_body,
    mesh=_mesh,
    out_type=jax.ShapeDtypeStruct((NC, NPAD), jnp.float32),
    scratch_types=[
        pltpu.VMEM((CH, LANES), jnp.int32),
        pltpu.VMEM((LANES,), jnp.float32),
        pltpu.VMEM((RPS,), jnp.float32),
        pltpu.VMEM_SHARED((NPAD,), jnp.float32),
    ],
)


def _seg_body(table_hbm, src_hbm, dst_hbm, out_hbm, src_idx, dst_idx, rows, acc,
              sem):
    cid = lax.axis_index("c")
    sid = lax.axis_index("s")
    wid = sid * NC + cid

    def fill(i, carry):
        for k in range(DT // 16):
            rows[i, pl.ds(k * 16, 16)] = jnp.zeros((16,), jnp.float32)
        return carry

    lax.fori_loop(0, LANES, fill, 0)
    for k in range(RPS // LANES):
        pltpu.sync_copy(rows, acc.at[pl.ds(sid * RPS + k * LANES, LANES)])
    plsc.subcore_barrier()

    pltpu.sync_copy(src_hbm.at[wid], src_idx)
    pltpu.sync_copy(dst_hbm.at[wid], dst_idx)

    def step(j, carry):
        pltpu.async_copy(table_hbm.at[src_idx.at[j]], rows, sem).wait()
        pltpu.sync_copy(rows, acc.at[dst_idx.at[j]], add=True)
        return carry

    lax.fori_loop(0, CH, step, 0)
    plsc.subcore_barrier()

    for k in range(RPS // LANES):
        s = sid * RPS + k * LANES
        pltpu.sync_copy(acc.at[pl.ds(s, LANES)], rows)
        pltpu.sync_copy(rows, out_hbm.at[cid, pl.ds(s, LANES)])


_seg_kernel = pl.kernel(
    _seg_body,
    mesh=_mesh,
    out_type=jax.ShapeDtypeStruct((NC, NPAD, DT), jnp.float32),
    scratch_types=[
        pltpu.VMEM((CH, LANES), jnp.int32),
        pltpu.VMEM((CH, LANES), jnp.int32),
        pltpu.VMEM((LANES, DT), jnp.float32),
        pltpu.VMEM_SHARED((NPAD, DT), jnp.float32),
        pltpu.SemaphoreType.DMA,
    ],
)


def _mm_body(x_ref, w_ref, o_ref):
    o_ref[...] = lax.dot_general(
        x_ref[...], w_ref[...], (((1,), (0,)), ((), ())),
        preferred_element_type=jnp.float32,
        precision=lax.Precision.HIGHEST)


_mm = pl.pallas_call(
    _mm_body,
    grid=(GRID,),
    in_specs=[
        pl.BlockSpec((BLK, DIN), lambda i: (i, 0)),
        pl.BlockSpec((DIN, D), lambda i: (0, 0)),
    ],
    out_specs=pl.BlockSpec((BLK, D), lambda i: (i, 0)),
    out_shape=jax.ShapeDtypeStruct((NPAD, D), jnp.float32),
)


def _scale_body(deg_ref, xw_ref, t_ref, dinv_ref):
    degt = deg_ref[0] + deg_ref[1] + 1.0
    dinv = lax.rsqrt(degt)
    dinv_ref[...] = dinv
    t_ref[:, 0:D] = xw_ref[...] * dinv
    t_ref[:, D:DT] = jnp.zeros((BLK, DT - D), jnp.float32)


_scale = pl.pallas_call(
    _scale_body,
    grid=(GRID,),
    in_specs=[
        pl.BlockSpec((NC, BLK, 1), lambda i: (0, i, 0)),
        pl.BlockSpec((BLK, D), lambda i: (i, 0)),
    ],
    out_specs=[
        pl.BlockSpec((BLK, DT), lambda i: (i, 0)),
        pl.BlockSpec((BLK, 1), lambda i: (i, 0)),
    ],
    out_shape=[
        jax.ShapeDtypeStruct((NPAD, DT), jnp.float32),
        jax.ShapeDtypeStruct((NPAD, 1), jnp.float32),
    ],
)


def _layer_body(t1_ref, seg_ref, dinv_ref, b_ref, w_ref, t2_ref):
    dinv = dinv_ref[...]
    seg = seg_ref[0, :, 0:D] + seg_ref[1, :, 0:D]
    pre = (seg + t1_ref[:, 0:D]) * dinv + b_ref[...][None, :]
    h = jnp.maximum(pre, 0.0)
    xw2 = lax.dot_general(
        h, w_ref[...], (((1,), (0,)), ((), ())),
        preferred_element_type=jnp.float32,
        precision=lax.Precision.HIGHEST)
    t2_ref[:, 0:D] = xw2 * dinv
    t2_ref[:, D:DT] = jnp.zeros((BLK, DT - D), jnp.float32)


_layer = pl.pallas_call(
    _layer_body,
    grid=(GRID,),
    in_specs=[
        pl.BlockSpec((BLK, DT), lambda i: (i, 0)),
        pl.BlockSpec((NC, BLK, DT), lambda i: (0, i, 0)),
        pl.BlockSpec((BLK, 1), lambda i: (i, 0)),
        pl.BlockSpec((D,), lambda i: (0,)),
        pl.BlockSpec((D, D), lambda i: (0, 0)),
    ],
    out_specs=pl.BlockSpec((BLK, DT), lambda i: (i, 0)),
    out_shape=jax.ShapeDtypeStruct((NPAD, DT), jnp.float32),
)


def _final_body(t2_ref, seg_ref, dinv_ref, b_ref, o_ref):
    i = pl.program_id(0)
    seg = seg_ref[0, :, 0:D] + seg_ref[1, :, 0:D]
    pre = (seg + t2_ref[:, 0:D]) * dinv_ref[...] + b_ref[...][None, :]
    h = jnp.maximum(pre, 0.0)
    row = lax.broadcasted_iota(jnp.int32, (BLK, 1), 0) + i * BLK
    h = jnp.where(row < N, h, 0.0)
    s = jnp.sum(h, axis=0) * (1.0 / N)

    @pl.when(i == 0)
    def _init():
        o_ref[...] = s

    @pl.when(i > 0)
    def _acc():
        o_ref[...] = o_ref[...] + s


_final = pl.pallas_call(
    _final_body,
    grid=(GRID,),
    in_specs=[
        pl.BlockSpec((BLK, DT), lambda i: (i, 0)),
        pl.BlockSpec((NC, BLK, DT), lambda i: (0, i, 0)),
        pl.BlockSpec((BLK, 1), lambda i: (i, 0)),
        pl.BlockSpec((D,), lambda i: (0,)),
    ],
    out_specs=pl.BlockSpec((D,), lambda i: (0,)),
    out_shape=jax.ShapeDtypeStruct((D,), jnp.float32),
)


def kernel(x, edge_index, W1, b1, W2, b2):
    src = edge_index[0]
    dst = edge_index[1]
    pad = jnp.full((EPAD - E,), N, jnp.int32)
    srcr = jnp.concatenate([src, pad]).reshape(NW, CH, LANES)
    dstr = jnp.concatenate([dst, pad]).reshape(NW, CH, LANES)
    xp = jnp.pad(x, ((0, NPAD - N), (0, 0)))

    deg = _deg_kernel(dstr)
    xw1 = _mm(xp, W1)
    t1, dinv = _scale(jnp.reshape(deg, (NC, NPAD, 1)), xw1)
    seg1 = _seg_kernel(t1, srcr, dstr)
    t2 = _layer(t1, seg1, dinv, b1, W2)
    seg2 = _seg_kernel(t2, srcr, dstr)
    return _final(t2, seg2, dinv, b2)


# speedup vs baseline: 18.9289x; 1.6276x over previous
import functools

import jax
import jax.numpy as jnp
from jax import lax
from jax.experimental import pallas as pl
from jax.experimental.pallas import tpu as pltpu
from jax.experimental.pallas import tpu_sc as plsc

N = 10000
NPAD = 10240
DIN = 128
D = 64
DT = 128
E = 320000
LANES = 128
CH = 79
NC = 2
NS = 16
NW = NC * NS
EPAD = NW * CH * LANES
RPS = NPAD // NS
BLK = 512
GRID = NPAD // BLK

_mesh = plsc.VectorSubcoreMesh(core_axis_name="c", subcore_axis_name="s")


def _deg_body(dst_hbm, out_hbm, dst_idx, ones_v, zbuf, acc):
    cid = lax.axis_index("c")
    sid = lax.axis_index("s")
    wid = sid * NC + cid

    def fill(i, carry):
        zbuf[pl.ds(i * 16, 16)] = jnp.zeros((16,), jnp.float32)
        return carry

    lax.fori_loop(0, RPS // 16, fill, 0)
    for k in range(LANES // 16):
        ones_v[pl.ds(k * 16, 16)] = jnp.ones((16,), jnp.float32)

    pltpu.sync_copy(zbuf, acc.at[pl.ds(sid * RPS, RPS)])
    plsc.subcore_barrier()

    pltpu.sync_copy(dst_hbm.at[wid], dst_idx)

    def step(j, carry):
        pltpu.sync_copy(ones_v, acc.at[dst_idx.at[j]], add=True)
        return carry

    lax.fori_loop(0, CH, step, 0)
    plsc.subcore_barrier()

    pltpu.sync_copy(acc.at[pl.ds(sid * RPS, RPS)], zbuf)
    pltpu.sync_copy(zbuf, out_hbm.at[cid, pl.ds(sid * RPS, RPS)])


_deg_kernel = pl.kernel(
    _deg_body,
    mesh=_mesh,
    out_type=jax.ShapeDtypeStruct((NC, NPAD), jnp.float32),
    scratch_types=[
        pltpu.VMEM((CH, LANES), jnp.int32),
        pltpu.VMEM((LANES,), jnp.float32),
        pltpu.VMEM((RPS,), jnp.float32),
        pltpu.VMEM_SHARED((NPAD,), jnp.float32),
    ],
)


def _seg_body(table_hbm, src_hbm, dst_hbm, out_hbm, src_idx, dst_idx, rows, acc,
              sem):
    cid = lax.axis_index("c")
    sid = lax.axis_index("s")
    wid = sid * NC + cid

    def fill(i, carry):
        for k in range(D // 16):
            rows[i, pl.ds(k * 16, 16)] = jnp.zeros((16,), jnp.float32)
        return carry

    lax.fori_loop(0, LANES, fill, 0)
    for k in range(RPS // LANES):
        pltpu.sync_copy(rows, acc.at[pl.ds(sid * RPS + k * LANES, LANES)])
    plsc.subcore_barrier()

    pltpu.sync_copy(src_hbm.at[wid], src_idx)
    pltpu.sync_copy(dst_hbm.at[wid], dst_idx)

    def step(j, carry):
        pltpu.async_copy(table_hbm.at[src_idx.at[j]], rows, sem).wait()
        pltpu.sync_copy(rows, acc.at[dst_idx.at[j]], add=True)
        return carry

    lax.fori_loop(0, CH, step, 0)
    plsc.subcore_barrier()

    for k in range(RPS // LANES):
        s = sid * RPS + k * LANES
        pltpu.sync_copy(acc.at[pl.ds(s, LANES)], rows)
        pltpu.sync_copy(rows, out_hbm.at[cid, pl.ds(s, LANES)])


_seg_kernel = pl.kernel(
    _seg_body,
    mesh=_mesh,
    out_type=jax.ShapeDtypeStruct((NC, NPAD, D), jnp.float32),
    scratch_types=[
        pltpu.VMEM((CH, LANES), jnp.int32),
        pltpu.VMEM((CH, LANES), jnp.int32),
        pltpu.VMEM((LANES, D), jnp.float32),
        pltpu.VMEM_SHARED((NPAD, D), jnp.float32),
        pltpu.SemaphoreType.DMA,
    ],
    compiler_params=pltpu.CompilerParams(use_tc_tiling_on_sc=False),
)


def _mm_body(x_ref, w_ref, o_ref):
    o_ref[...] = lax.dot_general(
        x_ref[...], w_ref[...], (((1,), (0,)), ((), ())),
        preferred_element_type=jnp.float32,
        precision=lax.Precision.HIGHEST)


_mm = pl.pallas_call(
    _mm_body,
    grid=(GRID,),
    in_specs=[
        pl.BlockSpec((BLK, DIN), lambda i: (i, 0)),
        pl.BlockSpec((DIN, D), lambda i: (0, 0)),
    ],
    out_specs=pl.BlockSpec((BLK, D), lambda i: (i, 0)),
    out_shape=jax.ShapeDtypeStruct((NPAD, D), jnp.float32),
)


def _scale_body(deg_ref, xw_ref, t_ref, dinv_ref):
    degt = deg_ref[0] + deg_ref[1] + 1.0
    dinv = lax.rsqrt(degt)
    dinv_ref[...] = dinv
    t_ref[...] = xw_ref[...] * dinv


_scale = pl.pallas_call(
    _scale_body,
    grid=(GRID,),
    in_specs=[
        pl.BlockSpec((NC, BLK, 1), lambda i: (0, i, 0)),
        pl.BlockSpec((BLK, D), lambda i: (i, 0)),
    ],
    out_specs=[
        pl.BlockSpec((BLK, D), lambda i: (i, 0)),
        pl.BlockSpec((BLK, 1), lambda i: (i, 0)),
    ],
    out_shape=[
        jax.ShapeDtypeStruct((NPAD, D), jnp.float32),
        jax.ShapeDtypeStruct((NPAD, 1), jnp.float32),
    ],
)


def _layer_body(t1_ref, seg_ref, dinv_ref, b_ref, w_ref, t2_ref):
    dinv = dinv_ref[...]
    seg = seg_ref[0] + seg_ref[1]
    pre = (seg + t1_ref[...]) * dinv + b_ref[...][None, :]
    h = jnp.maximum(pre, 0.0)
    xw2 = lax.dot_general(
        h, w_ref[...], (((1,), (0,)), ((), ())),
        preferred_element_type=jnp.float32,
        precision=lax.Precision.HIGHEST)
    t2_ref[...] = xw2 * dinv


_layer = pl.pallas_call(
    _layer_body,
    grid=(GRID,),
    in_specs=[
        pl.BlockSpec((BLK, D), lambda i: (i, 0)),
        pl.BlockSpec((NC, BLK, D), lambda i: (0, i, 0)),
        pl.BlockSpec((BLK, 1), lambda i: (i, 0)),
        pl.BlockSpec((D,), lambda i: (0,)),
        pl.BlockSpec((D, D), lambda i: (0, 0)),
    ],
    out_specs=pl.BlockSpec((BLK, D), lambda i: (i, 0)),
    out_shape=jax.ShapeDtypeStruct((NPAD, D), jnp.float32),
)


def _final_body(t2_ref, seg_ref, dinv_ref, b_ref, o_ref):
    i = pl.program_id(0)
    seg = seg_ref[0] + seg_ref[1]
    pre = (seg + t2_ref[...]) * dinv_ref[...] + b_ref[...][None, :]
    h = jnp.maximum(pre, 0.0)
    row = lax.broadcasted_iota(jnp.int32, (BLK, 1), 0) + i * BLK
    h = jnp.where(row < N, h, 0.0)
    s = jnp.sum(h, axis=0) * (1.0 / N)

    @pl.when(i == 0)
    def _init():
        o_ref[...] = s

    @pl.when(i > 0)
    def _acc():
        o_ref[...] = o_ref[...] + s


_final = pl.pallas_call(
    _final_body,
    grid=(GRID,),
    in_specs=[
        pl.BlockSpec((BLK, D), lambda i: (i, 0)),
        pl.BlockSpec((NC, BLK, D), lambda i: (0, i, 0)),
        pl.BlockSpec((BLK, 1), lambda i: (i, 0)),
        pl.BlockSpec((D,), lambda i: (0,)),
    ],
    out_specs=pl.BlockSpec((D,), lambda i: (0,)),
    out_shape=jax.ShapeDtypeStruct((D,), jnp.float32),
)


def kernel(x, edge_index, W1, b1, W2, b2):
    src = edge_index[0]
    dst = edge_index[1]
    pad = jnp.full((EPAD - E,), N, jnp.int32)
    srcr = jnp.concatenate([src, pad]).reshape(NW, CH, LANES)
    dstr = jnp.concatenate([dst, pad]).reshape(NW, CH, LANES)
    xp = jnp.pad(x, ((0, NPAD - N), (0, 0)))

    deg = _deg_kernel(dstr)
    xw1 = _mm(xp, W1)
    t1, dinv = _scale(jnp.reshape(deg, (NC, NPAD, 1)), xw1)
    seg1 = _seg_kernel(t1, srcr, dstr)
    t2 = _layer(t1, seg1, dinv, b1, W2)
    seg2 = _seg_kernel(t2, srcr, dstr)
    return _final(t2, seg2, dinv, b2)


# speedup vs baseline: 32.1582x; 1.6989x over previous
import functools

import jax
import jax.numpy as jnp
from jax import lax
from jax.experimental import pallas as pl
from jax.experimental.pallas import tpu as pltpu
from jax.experimental.pallas import tpu_sc as plsc

N = 10000
NPAD = 10240
DIN = 128
D = 64
DT = 128
E = 320000
LANES = 128
CH = 80
NC = 2
NS = 16
NW = NC * NS
EPAD = NW * CH * LANES
RPS = NPAD // NS
BLK = 512
GRID = NPAD // BLK

_mesh = plsc.VectorSubcoreMesh(core_axis_name="c", subcore_axis_name="s")


def _deg_body(dst_hbm, out_hbm, dst_idx, ones_v, zbuf, acc):
    cid = lax.axis_index("c")
    sid = lax.axis_index("s")
    wid = sid * NC + cid

    def fill(i, carry):
        zbuf[pl.ds(i * 16, 16)] = jnp.zeros((16,), jnp.float32)
        return carry

    lax.fori_loop(0, RPS // 16, fill, 0)
    for k in range(LANES // 16):
        ones_v[pl.ds(k * 16, 16)] = jnp.ones((16,), jnp.float32)

    pltpu.sync_copy(zbuf, acc.at[pl.ds(sid * RPS, RPS)])
    plsc.subcore_barrier()

    pltpu.sync_copy(dst_hbm.at[wid], dst_idx)

    def step(j, carry):
        pltpu.sync_copy(ones_v, acc.at[dst_idx.at[j]], add=True)
        return carry

    lax.fori_loop(0, CH, step, 0)
    plsc.subcore_barrier()

    pltpu.sync_copy(acc.at[pl.ds(sid * RPS, RPS)], zbuf)
    pltpu.sync_copy(zbuf, out_hbm.at[cid, pl.ds(sid * RPS, RPS)])


_deg_kernel = pl.kernel(
    _deg_body,
    mesh=_mesh,
    out_type=jax.ShapeDtypeStruct((NC, NPAD), jnp.float32),
    scratch_types=[
        pltpu.VMEM((CH, LANES), jnp.int32),
        pltpu.VMEM((LANES,), jnp.float32),
        pltpu.VMEM((RPS,), jnp.float32),
        pltpu.VMEM_SHARED((NPAD,), jnp.float32),
    ],
)


def _seg_body(table_hbm, src_hbm, dst_hbm, out_hbm, src_idx, dst_idx, rows0,
              rows1, acc, g0, g1, s0, s1):
    cid = lax.axis_index("c")
    sid = lax.axis_index("s")
    wid = sid * NC + cid

    def fill(i, carry):
        for k in range(D // 16):
            rows0[i, pl.ds(k * 16, 16)] = jnp.zeros((16,), jnp.float32)
        return carry

    lax.fori_loop(0, LANES, fill, 0)
    for k in range(RPS // LANES):
        pltpu.sync_copy(rows0, acc.at[pl.ds(sid * RPS + k * LANES, LANES)])
    plsc.subcore_barrier()

    pltpu.sync_copy(src_hbm.at[wid], src_idx)
    pltpu.sync_copy(dst_hbm.at[wid], dst_idx)

    pltpu.async_copy(table_hbm.at[src_idx.at[0]], rows0, g0)

    def step(i, carry):
        j0 = 2 * i
        pltpu.make_async_copy(table_hbm.at[src_idx.at[j0]], rows0, g0).wait()
        pltpu.async_copy(rows0, acc.at[dst_idx.at[j0]], s0, add=True)

        @pl.when(i > 0)
        def _w1():
            pltpu.make_async_copy(rows1, acc.at[dst_idx.at[j0 - 1]], s1).wait()

        pltpu.async_copy(table_hbm.at[src_idx.at[j0 + 1]], rows1, g1)

        pltpu.make_async_copy(rows0, acc.at[dst_idx.at[j0]], s0).wait()

        @pl.when(i + 1 < CH // 2)
        def _g0():
            pltpu.async_copy(table_hbm.at[src_idx.at[j0 + 2]], rows0, g0)

        pltpu.make_async_copy(table_hbm.at[src_idx.at[j0 + 1]], rows1, g1).wait()
        pltpu.async_copy(rows1, acc.at[dst_idx.at[j0 + 1]], s1, add=True)
        return carry

    lax.fori_loop(0, CH // 2, step, 0)
    pltpu.make_async_copy(rows1, acc.at[dst_idx.at[CH - 1]], s1).wait()
    plsc.subcore_barrier()

    for k in range(RPS // LANES):
        s = sid * RPS + k * LANES
        pltpu.sync_copy(acc.at[pl.ds(s, LANES)], rows0)
        pltpu.sync_copy(rows0, out_hbm.at[cid, pl.ds(s, LANES)])


_seg_kernel = pl.kernel(
    _seg_body,
    mesh=_mesh,
    out_type=jax.ShapeDtypeStruct((NC, NPAD, D), jnp.float32),
    scratch_types=[
        pltpu.VMEM((CH, LANES), jnp.int32),
        pltpu.VMEM((CH, LANES), jnp.int32),
        pltpu.VMEM((LANES, D), jnp.float32),
        pltpu.VMEM((LANES, D), jnp.float32),
        pltpu.VMEM_SHARED((NPAD, D), jnp.float32),
        pltpu.SemaphoreType.DMA,
        pltpu.SemaphoreType.DMA,
        pltpu.SemaphoreType.DMA,
        pltpu.SemaphoreType.DMA,
    ],
    compiler_params=pltpu.CompilerParams(use_tc_tiling_on_sc=False),
)


def _mm_body(x_ref, w_ref, o_ref):
    o_ref[...] = lax.dot_general(
        x_ref[...], w_ref[...], (((1,), (0,)), ((), ())),
        preferred_element_type=jnp.float32,
        precision=lax.Precision.HIGHEST)


_mm = pl.pallas_call(
    _mm_body,
    grid=(GRID,),
    in_specs=[
        pl.BlockSpec((BLK, DIN), lambda i: (i, 0)),
        pl.BlockSpec((DIN, D), lambda i: (0, 0)),
    ],
    out_specs=pl.BlockSpec((BLK, D), lambda i: (i, 0)),
    out_shape=jax.ShapeDtypeStruct((NPAD, D), jnp.float32),
)


def _scale_body(deg_ref, xw_ref, t_ref, dinv_ref):
    degt = deg_ref[0] + deg_ref[1] + 1.0
    dinv = lax.rsqrt(degt)
    dinv_ref[...] = dinv
    t_ref[...] = xw_ref[...] * dinv


_scale = pl.pallas_call(
    _scale_body,
    grid=(GRID,),
    in_specs=[
        pl.BlockSpec((NC, BLK, 1), lambda i: (0, i, 0)),
        pl.BlockSpec((BLK, D), lambda i: (i, 0)),
    ],
    out_specs=[
        pl.BlockSpec((BLK, D), lambda i: (i, 0)),
        pl.BlockSpec((BLK, 1), lambda i: (i, 0)),
    ],
    out_shape=[
        jax.ShapeDtypeStruct((NPAD, D), jnp.float32),
        jax.ShapeDtypeStruct((NPAD, 1), jnp.float32),
    ],
)


def _layer_body(t1_ref, seg_ref, dinv_ref, b_ref, w_ref, t2_ref):
    dinv = dinv_ref[...]
    seg = seg_ref[0] + seg_ref[1]
    pre = (seg + t1_ref[...]) * dinv + b_ref[...][None, :]
    h = jnp.maximum(pre, 0.0)
    xw2 = lax.dot_general(
        h, w_ref[...], (((1,), (0,)), ((), ())),
        preferred_element_type=jnp.float32,
        precision=lax.Precision.HIGHEST)
    t2_ref[...] = xw2 * dinv


_layer = pl.pallas_call(
    _layer_body,
    grid=(GRID,),
    in_specs=[
        pl.BlockSpec((BLK, D), lambda i: (i, 0)),
        pl.BlockSpec((NC, BLK, D), lambda i: (0, i, 0)),
        pl.BlockSpec((BLK, 1), lambda i: (i, 0)),
        pl.BlockSpec((D,), lambda i: (0,)),
        pl.BlockSpec((D, D), lambda i: (0, 0)),
    ],
    out_specs=pl.BlockSpec((BLK, D), lambda i: (i, 0)),
    out_shape=jax.ShapeDtypeStruct((NPAD, D), jnp.float32),
)


def _final_body(t2_ref, seg_ref, dinv_ref, b_ref, o_ref):
    i = pl.program_id(0)
    seg = seg_ref[0] + seg_ref[1]
    pre = (seg + t2_ref[...]) * dinv_ref[...] + b_ref[...][None, :]
    h = jnp.maximum(pre, 0.0)
    row = lax.broadcasted_iota(jnp.int32, (BLK, 1), 0) + i * BLK
    h = jnp.where(row < N, h, 0.0)
    s = jnp.sum(h, axis=0) * (1.0 / N)

    @pl.when(i == 0)
    def _init():
        o_ref[...] = s

    @pl.when(i > 0)
    def _acc():
        o_ref[...] = o_ref[...] + s


_final = pl.pallas_call(
    _final_body,
    grid=(GRID,),
    in_specs=[
        pl.BlockSpec((BLK, D), lambda i: (i, 0)),
        pl.BlockSpec((NC, BLK, D), lambda i: (0, i, 0)),
        pl.BlockSpec((BLK, 1), lambda i: (i, 0)),
        pl.BlockSpec((D,), lambda i: (0,)),
    ],
    out_specs=pl.BlockSpec((D,), lambda i: (0,)),
    out_shape=jax.ShapeDtypeStruct((D,), jnp.float32),
)


def kernel(x, edge_index, W1, b1, W2, b2):
    src = edge_index[0]
    dst = edge_index[1]
    pad = (jnp.arange(EPAD - E, dtype=jnp.int32) % (NPAD - N)) + N
    srcr = jnp.concatenate([src, pad]).reshape(NW, CH, LANES)
    dstr = jnp.concatenate([dst, pad]).reshape(NW, CH, LANES)
    xp = jnp.pad(x, ((0, NPAD - N), (0, 0)))

    deg = _deg_kernel(dstr)
    xw1 = _mm(xp, W1)
    t1, dinv = _scale(jnp.reshape(deg, (NC, NPAD, 1)), xw1)
    seg1 = _seg_kernel(t1, srcr, dstr)
    t2 = _layer(t1, seg1, dinv, b1, W2)
    seg2 = _seg_kernel(t2, srcr, dstr)
    return _final(t2, seg2, dinv, b2)


# speedup vs baseline: 38.2229x; 1.1886x over previous
import functools

import jax
import jax.numpy as jnp
from jax import lax
from jax.experimental import pallas as pl
from jax.experimental.pallas import tpu as pltpu
from jax.experimental.pallas import tpu_sc as plsc

N = 10000
NPAD = 10240
DIN = 128
D = 64
DT = 128
E = 320000
LANES = 128
CH = 80
NC = 2
NS = 16
NW = NC * NS
EPAD = NW * CH * LANES
RPS = NPAD // NS
BLK = 512
GRID = NPAD // BLK

_mesh = plsc.VectorSubcoreMesh(core_axis_name="c", subcore_axis_name="s")


def _deg_body(dst_hbm, out_hbm, dst_idx, ones_v, zbuf, acc):
    cid = lax.axis_index("c")
    sid = lax.axis_index("s")
    wid = sid * NC + cid

    def fill(i, carry):
        zbuf[pl.ds(i * 16, 16)] = jnp.zeros((16,), jnp.float32)
        return carry

    lax.fori_loop(0, RPS // 16, fill, 0)
    for k in range(LANES // 16):
        ones_v[pl.ds(k * 16, 16)] = jnp.ones((16,), jnp.float32)

    pltpu.sync_copy(zbuf, acc.at[pl.ds(sid * RPS, RPS)])
    plsc.subcore_barrier()

    pltpu.sync_copy(dst_hbm.at[wid], dst_idx)

    def step(j, carry):
        pltpu.sync_copy(ones_v, acc.at[dst_idx.at[j]], add=True)
        return carry

    lax.fori_loop(0, CH, step, 0)
    plsc.subcore_barrier()

    pltpu.sync_copy(acc.at[pl.ds(sid * RPS, RPS)], zbuf)
    pltpu.sync_copy(zbuf, out_hbm.at[cid, pl.ds(sid * RPS, RPS)])


_deg_kernel = pl.kernel(
    _deg_body,
    mesh=_mesh,
    out_type=jax.ShapeDtypeStruct((NC, NPAD), jnp.float32),
    scratch_types=[
        pltpu.VMEM((CH, LANES), jnp.int32),
        pltpu.VMEM((LANES,), jnp.float32),
        pltpu.VMEM((RPS,), jnp.float32),
        pltpu.VMEM_SHARED((NPAD,), jnp.float32),
    ],
)


NBUF = 4


def _seg_body(table_hbm, src_hbm, dst_hbm, out_hbm, src_idx, dst_idx, rows,
              acc, gsems, ssems):
    cid = lax.axis_index("c")
    sid = lax.axis_index("s")
    wid = sid * NC + cid

    def fill(i, carry):
        for k in range(D // 16):
            rows[0][i, pl.ds(k * 16, 16)] = jnp.zeros((16,), jnp.float32)
        return carry

    lax.fori_loop(0, LANES, fill, 0)
    for k in range(RPS // LANES):
        pltpu.sync_copy(rows[0], acc.at[pl.ds(sid * RPS + k * LANES, LANES)])
    plsc.subcore_barrier()

    pltpu.sync_copy(src_hbm.at[wid], src_idx)
    pltpu.sync_copy(dst_hbm.at[wid], dst_idx)

    for b in range(NBUF):
        pltpu.async_copy(table_hbm.at[src_idx.at[b]], rows[b], gsems[b])

    def step(i, carry):
        j = i * NBUF
        for b in range(NBUF):
            pltpu.make_async_copy(
                table_hbm.at[src_idx.at[j + b]], rows[b], gsems[b]).wait()
            pltpu.async_copy(rows[b], acc.at[dst_idx.at[j + b]], ssems[b],
                             add=True)
        for b in range(NBUF):
            pltpu.make_async_copy(
                rows[b], acc.at[dst_idx.at[j + b]], ssems[b]).wait()

            @pl.when(i + 1 < CH // NBUF)
            def _g():
                pltpu.async_copy(table_hbm.at[src_idx.at[j + NBUF + b]],
                                 rows[b], gsems[b])
        return carry

    lax.fori_loop(0, CH // NBUF, step, 0)
    plsc.subcore_barrier()

    for k in range(RPS // LANES):
        s = sid * RPS + k * LANES
        pltpu.sync_copy(acc.at[pl.ds(s, LANES)], rows[0])
        pltpu.sync_copy(rows[0], out_hbm.at[cid, pl.ds(s, LANES)])


_seg_kernel = pl.kernel(
    _seg_body,
    mesh=_mesh,
    out_type=jax.ShapeDtypeStruct((NC, NPAD, D), jnp.float32),
    scratch_types=[
        pltpu.VMEM((CH, LANES), jnp.int32),
        pltpu.VMEM((CH, LANES), jnp.int32),
        [pltpu.VMEM((LANES, D), jnp.float32) for _ in range(NBUF)],
        pltpu.VMEM_SHARED((NPAD, D), jnp.float32),
        [pltpu.SemaphoreType.DMA for _ in range(NBUF)],
        [pltpu.SemaphoreType.DMA for _ in range(NBUF)],
    ],
    compiler_params=pltpu.CompilerParams(use_tc_tiling_on_sc=False),
)


def _mm_body(x_ref, w_ref, o_ref):
    o_ref[...] = lax.dot_general(
        x_ref[...], w_ref[...], (((1,), (0,)), ((), ())),
        preferred_element_type=jnp.float32,
        precision=lax.Precision.HIGHEST)


_mm = pl.pallas_call(
    _mm_body,
    grid=(GRID,),
    in_specs=[
        pl.BlockSpec((BLK, DIN), lambda i: (i, 0)),
        pl.BlockSpec((DIN, D), lambda i: (0, 0)),
    ],
    out_specs=pl.BlockSpec((BLK, D), lambda i: (i, 0)),
    out_shape=jax.ShapeDtypeStruct((NPAD, D), jnp.float32),
)


def _scale_body(deg_ref, xw_ref, t_ref, dinv_ref):
    degt = deg_ref[0] + deg_ref[1] + 1.0
    dinv = lax.rsqrt(degt)
    dinv_ref[...] = dinv
    t_ref[...] = xw_ref[...] * dinv


_scale = pl.pallas_call(
    _scale_body,
    grid=(GRID,),
    in_specs=[
        pl.BlockSpec((NC, BLK, 1), lambda i: (0, i, 0)),
        pl.BlockSpec((BLK, D), lambda i: (i, 0)),
    ],
    out_specs=[
        pl.BlockSpec((BLK, D), lambda i: (i, 0)),
        pl.BlockSpec((BLK, 1), lambda i: (i, 0)),
    ],
    out_shape=[
        jax.ShapeDtypeStruct((NPAD, D), jnp.float32),
        jax.ShapeDtypeStruct((NPAD, 1), jnp.float32),
    ],
)


def _layer_body(t1_ref, seg_ref, dinv_ref, b_ref, w_ref, t2_ref):
    dinv = dinv_ref[...]
    seg = seg_ref[0] + seg_ref[1]
    pre = (seg + t1_ref[...]) * dinv + b_ref[...][None, :]
    h = jnp.maximum(pre, 0.0)
    xw2 = lax.dot_general(
        h, w_ref[...], (((1,), (0,)), ((), ())),
        preferred_element_type=jnp.float32,
        precision=lax.Precision.HIGHEST)
    t2_ref[...] = xw2 * dinv


_layer = pl.pallas_call(
    _layer_body,
    grid=(GRID,),
    in_specs=[
        pl.BlockSpec((BLK, D), lambda i: (i, 0)),
        pl.BlockSpec((NC, BLK, D), lambda i: (0, i, 0)),
        pl.BlockSpec((BLK, 1), lambda i: (i, 0)),
        pl.BlockSpec((D,), lambda i: (0,)),
        pl.BlockSpec((D, D), lambda i: (0, 0)),
    ],
    out_specs=pl.BlockSpec((BLK, D), lambda i: (i, 0)),
    out_shape=jax.ShapeDtypeStruct((NPAD, D), jnp.float32),
)


def _final_body(t2_ref, seg_ref, dinv_ref, b_ref, o_ref):
    i = pl.program_id(0)
    seg = seg_ref[0] + seg_ref[1]
    pre = (seg + t2_ref[...]) * dinv_ref[...] + b_ref[...][None, :]
    h = jnp.maximum(pre, 0.0)
    row = lax.broadcasted_iota(jnp.int32, (BLK, 1), 0) + i * BLK
    h = jnp.where(row < N, h, 0.0)
    s = jnp.sum(h, axis=0) * (1.0 / N)

    @pl.when(i == 0)
    def _init():
        o_ref[...] = s

    @pl.when(i > 0)
    def _acc():
        o_ref[...] = o_ref[...] + s


_final = pl.pallas_call(
    _final_body,
    grid=(GRID,),
    in_specs=[
        pl.BlockSpec((BLK, D), lambda i: (i, 0)),
        pl.BlockSpec((NC, BLK, D), lambda i: (0, i, 0)),
        pl.BlockSpec((BLK, 1), lambda i: (i, 0)),
        pl.BlockSpec((D,), lambda i: (0,)),
    ],
    out_specs=pl.BlockSpec((D,), lambda i: (0,)),
    out_shape=jax.ShapeDtypeStruct((D,), jnp.float32),
)


def kernel(x, edge_index, W1, b1, W2, b2):
    src = edge_index[0]
    dst = edge_index[1]
    pad = (jnp.arange(EPAD - E, dtype=jnp.int32) % (NPAD - N)) + N
    srcr = jnp.concatenate([src, pad]).reshape(NW, CH, LANES)
    dstr = jnp.concatenate([dst, pad]).reshape(NW, CH, LANES)
    xp = jnp.pad(x, ((0, NPAD - N), (0, 0)))

    deg = _deg_kernel(dstr)
    xw1 = _mm(xp, W1)
    t1, dinv = _scale(jnp.reshape(deg, (NC, NPAD, 1)), xw1)
    seg1 = _seg_kernel(t1, srcr, dstr)
    t2 = _layer(t1, seg1, dinv, b1, W2)
    seg2 = _seg_kernel(t2, srcr, dstr)
    return _final(t2, seg2, dinv, b2)


# speedup vs baseline: 39.2595x; 1.0271x over previous
import functools

import jax
import jax.numpy as jnp
from jax import lax
from jax.experimental import pallas as pl
from jax.experimental.pallas import tpu as pltpu
from jax.experimental.pallas import tpu_sc as plsc

N = 10000
NPAD = 10240
DIN = 128
D = 64
DT = 128
E = 320000
LANES = 128
CH = 80
NC = 2
NS = 16
NW = NC * NS
EPAD = NW * CH * LANES
RPS = NPAD // NS
BLK = 512
GRID = NPAD // BLK

_mesh = plsc.VectorSubcoreMesh(core_axis_name="c", subcore_axis_name="s")


def _deg_body(dst_hbm, out_hbm, dst_idx, ones_v, zbuf, acc):
    cid = lax.axis_index("c")
    sid = lax.axis_index("s")
    wid = sid * NC + cid

    def fill(i, carry):
        zbuf[pl.ds(i * 16, 16)] = jnp.zeros((16,), jnp.float32)
        return carry

    lax.fori_loop(0, RPS // 16, fill, 0)
    for k in range(LANES // 16):
        ones_v[pl.ds(k * 16, 16)] = jnp.ones((16,), jnp.float32)

    pltpu.sync_copy(zbuf, acc.at[pl.ds(sid * RPS, RPS)])
    plsc.subcore_barrier()

    pltpu.sync_copy(dst_hbm.at[wid], dst_idx)

    def step(j, carry):
        pltpu.sync_copy(ones_v, acc.at[dst_idx.at[j]], add=True)
        return carry

    lax.fori_loop(0, CH, step, 0)
    plsc.subcore_barrier()

    pltpu.sync_copy(acc.at[pl.ds(sid * RPS, RPS)], zbuf)
    pltpu.sync_copy(zbuf, out_hbm.at[cid, pl.ds(sid * RPS, RPS)])


_deg_kernel = pl.kernel(
    _deg_body,
    mesh=_mesh,
    out_type=jax.ShapeDtypeStruct((NC, NPAD), jnp.float32),
    scratch_types=[
        pltpu.VMEM((CH, LANES), jnp.int32),
        pltpu.VMEM((LANES,), jnp.float32),
        pltpu.VMEM((RPS,), jnp.float32),
        pltpu.VMEM_SHARED((NPAD,), jnp.float32),
    ],
)


NBUF = 8


def _seg_body(table_hbm, src_hbm, dst_hbm, out_hbm, src_idx, dst_idx, rows,
              acc, gsems, ssems):
    cid = lax.axis_index("c")
    sid = lax.axis_index("s")
    wid = sid * NC + cid

    def fill(i, carry):
        for k in range(D // 16):
            rows[0][i, pl.ds(k * 16, 16)] = jnp.zeros((16,), jnp.float32)
        return carry

    lax.fori_loop(0, LANES, fill, 0)
    for k in range(RPS // LANES):
        pltpu.sync_copy(rows[0], acc.at[pl.ds(sid * RPS + k * LANES, LANES)])
    plsc.subcore_barrier()

    pltpu.sync_copy(src_hbm.at[wid], src_idx)
    pltpu.sync_copy(dst_hbm.at[wid], dst_idx)

    for b in range(NBUF):
        pltpu.async_copy(table_hbm.at[src_idx.at[b]], rows[b], gsems[b])

    def step(i, carry):
        j = i * NBUF
        for b in range(NBUF):
            pltpu.make_async_copy(
                table_hbm.at[src_idx.at[j + b]], rows[b], gsems[b]).wait()
            pltpu.async_copy(rows[b], acc.at[dst_idx.at[j + b]], ssems[b],
                             add=True)
        for b in range(NBUF):
            pltpu.make_async_copy(
                rows[b], acc.at[dst_idx.at[j + b]], ssems[b]).wait()

            @pl.when(i + 1 < CH // NBUF)
            def _g():
                pltpu.async_copy(table_hbm.at[src_idx.at[j + NBUF + b]],
                                 rows[b], gsems[b])
        return carry

    lax.fori_loop(0, CH // NBUF, step, 0)
    plsc.subcore_barrier()

    for k in range(RPS // LANES):
        s = sid * RPS + k * LANES
        pltpu.sync_copy(acc.at[pl.ds(s, LANES)], rows[0])
        pltpu.sync_copy(rows[0], out_hbm.at[cid, pl.ds(s, LANES)])


_seg_kernel = pl.kernel(
    _seg_body,
    mesh=_mesh,
    out_type=jax.ShapeDtypeStruct((NC, NPAD, D), jnp.float32),
    scratch_types=[
        pltpu.VMEM((CH, LANES), jnp.int32),
        pltpu.VMEM((CH, LANES), jnp.int32),
        [pltpu.VMEM((LANES, D), jnp.float32) for _ in range(NBUF)],
        pltpu.VMEM_SHARED((NPAD, D), jnp.float32),
        [pltpu.SemaphoreType.DMA for _ in range(NBUF)],
        [pltpu.SemaphoreType.DMA for _ in range(NBUF)],
    ],
    compiler_params=pltpu.CompilerParams(use_tc_tiling_on_sc=False),
)


def _mm_body(x_ref, w_ref, o_ref):
    o_ref[...] = lax.dot_general(
        x_ref[...], w_ref[...], (((1,), (0,)), ((), ())),
        preferred_element_type=jnp.float32,
        precision=lax.Precision.HIGHEST)


_mm = pl.pallas_call(
    _mm_body,
    grid=(GRID,),
    in_specs=[
        pl.BlockSpec((BLK, DIN), lambda i: (i, 0)),
        pl.BlockSpec((DIN, D), lambda i: (0, 0)),
    ],
    out_specs=pl.BlockSpec((BLK, D), lambda i: (i, 0)),
    out_shape=jax.ShapeDtypeStruct((NPAD, D), jnp.float32),
)


def _scale_body(deg_ref, xw_ref, t_ref, dinv_ref):
    degt = deg_ref[0] + deg_ref[1] + 1.0
    dinv = lax.rsqrt(degt)
    dinv_ref[...] = dinv
    t_ref[...] = xw_ref[...] * dinv


_scale = pl.pallas_call(
    _scale_body,
    grid=(GRID,),
    in_specs=[
        pl.BlockSpec((NC, BLK, 1), lambda i: (0, i, 0)),
        pl.BlockSpec((BLK, D), lambda i: (i, 0)),
    ],
    out_specs=[
        pl.BlockSpec((BLK, D), lambda i: (i, 0)),
        pl.BlockSpec((BLK, 1), lambda i: (i, 0)),
    ],
    out_shape=[
        jax.ShapeDtypeStruct((NPAD, D), jnp.float32),
        jax.ShapeDtypeStruct((NPAD, 1), jnp.float32),
    ],
)


def _layer_body(t1_ref, seg_ref, dinv_ref, b_ref, w_ref, t2_ref):
    dinv = dinv_ref[...]
    seg = seg_ref[0] + seg_ref[1]
    pre = (seg + t1_ref[...]) * dinv + b_ref[...][None, :]
    h = jnp.maximum(pre, 0.0)
    xw2 = lax.dot_general(
        h, w_ref[...], (((1,), (0,)), ((), ())),
        preferred_element_type=jnp.float32,
        precision=lax.Precision.HIGHEST)
    t2_ref[...] = xw2 * dinv


_layer = pl.pallas_call(
    _layer_body,
    grid=(GRID,),
    in_specs=[
        pl.BlockSpec((BLK, D), lambda i: (i, 0)),
        pl.BlockSpec((NC, BLK, D), lambda i: (0, i, 0)),
        pl.BlockSpec((BLK, 1), lambda i: (i, 0)),
        pl.BlockSpec((D,), lambda i: (0,)),
        pl.BlockSpec((D, D), lambda i: (0, 0)),
    ],
    out_specs=pl.BlockSpec((BLK, D), lambda i: (i, 0)),
    out_shape=jax.ShapeDtypeStruct((NPAD, D), jnp.float32),
)


def _final_body(t2_ref, seg_ref, dinv_ref, b_ref, o_ref):
    i = pl.program_id(0)
    seg = seg_ref[0] + seg_ref[1]
    pre = (seg + t2_ref[...]) * dinv_ref[...] + b_ref[...][None, :]
    h = jnp.maximum(pre, 0.0)
    row = lax.broadcasted_iota(jnp.int32, (BLK, 1), 0) + i * BLK
    h = jnp.where(row < N, h, 0.0)
    s = jnp.sum(h, axis=0) * (1.0 / N)

    @pl.when(i == 0)
    def _init():
        o_ref[...] = s

    @pl.when(i > 0)
    def _acc():
        o_ref[...] = o_ref[...] + s


_final = pl.pallas_call(
    _final_body,
    grid=(GRID,),
    in_specs=[
        pl.BlockSpec((BLK, D), lambda i: (i, 0)),
        pl.BlockSpec((NC, BLK, D), lambda i: (0, i, 0)),
        pl.BlockSpec((BLK, 1), lambda i: (i, 0)),
        pl.BlockSpec((D,), lambda i: (0,)),
    ],
    out_specs=pl.BlockSpec((D,), lambda i: (0,)),
    out_shape=jax.ShapeDtypeStruct((D,), jnp.float32),
)


def kernel(x, edge_index, W1, b1, W2, b2):
    src = edge_index[0]
    dst = edge_index[1]
    pad = (jnp.arange(EPAD - E, dtype=jnp.int32) % (NPAD - N)) + N
    srcr = jnp.concatenate([src, pad]).reshape(NW, CH, LANES)
    dstr = jnp.concatenate([dst, pad]).reshape(NW, CH, LANES)
    xp = jnp.pad(x, ((0, NPAD - N), (0, 0)))

    deg = _deg_kernel(dstr)
    xw1 = _mm(xp, W1)
    t1, dinv = _scale(jnp.reshape(deg, (NC, NPAD, 1)), xw1)
    seg1 = _seg_kernel(t1, srcr, dstr)
    t2 = _layer(t1, seg1, dinv, b1, W2)
    seg2 = _seg_kernel(t2, srcr, dstr)
    return _final(t2, seg2, dinv, b2)


# speedup vs baseline: 40.3111x; 1.0268x over previous
import functools

import jax
import jax.numpy as jnp
from jax import lax
from jax.experimental import pallas as pl
from jax.experimental.pallas import tpu as pltpu
from jax.experimental.pallas import tpu_sc as plsc

N = 10000
NPAD = 10240
DIN = 128
D = 64
DT = 128
E = 320000
LANES = 128
CH = 80
NC = 2
NS = 16
NW = NC * NS
EPAD = NW * CH * LANES
RPS = NPAD // NS
BLK = 512
GRID = NPAD // BLK

_mesh = plsc.VectorSubcoreMesh(core_axis_name="c", subcore_axis_name="s")


NBUF = 8


def _seg_body(table_hbm, edges_hbm, out_hbm, src_idx, dst_idx, rows,
              acc, gsems, ssems):
    cid = lax.axis_index("c")
    sid = lax.axis_index("s")
    wid = sid * NC + cid

    pltpu.async_copy(edges_hbm.at[0, wid], src_idx, gsems[0])
    pltpu.async_copy(edges_hbm.at[1, wid], dst_idx, gsems[1])

    def fill(i, carry):
        for k in range(D // 16):
            rows[0][i, pl.ds(k * 16, 16)] = jnp.zeros((16,), jnp.float32)
        return carry

    lax.fori_loop(0, LANES, fill, 0)
    for k in range(RPS // LANES):
        pltpu.sync_copy(rows[0], acc.at[pl.ds(sid * RPS + k * LANES, LANES)])
    plsc.subcore_barrier()

    pltpu.make_async_copy(edges_hbm.at[0, wid], src_idx, gsems[0]).wait()
    pltpu.make_async_copy(edges_hbm.at[1, wid], dst_idx, gsems[1]).wait()

    for b in range(NBUF):
        pltpu.async_copy(table_hbm.at[src_idx.at[b]], rows[b], gsems[b])

    def step(i, carry):
        j = i * NBUF
        for b in range(NBUF):
            pltpu.make_async_copy(
                table_hbm.at[src_idx.at[j + b]], rows[b], gsems[b]).wait()
            pltpu.async_copy(rows[b], acc.at[dst_idx.at[j + b]], ssems[b],
                             add=True)
        for b in range(NBUF):
            pltpu.make_async_copy(
                rows[b], acc.at[dst_idx.at[j + b]], ssems[b]).wait()

            @pl.when(i + 1 < CH // NBUF)
            def _g():
                pltpu.async_copy(table_hbm.at[src_idx.at[j + NBUF + b]],
                                 rows[b], gsems[b])
        return carry

    lax.fori_loop(0, CH // NBUF, step, 0)
    plsc.subcore_barrier()

    pltpu.sync_copy(acc.at[pl.ds(sid * RPS, RPS)],
                    out_hbm.at[cid, pl.ds(sid * RPS, RPS)])


_seg_kernel = pl.kernel(
    _seg_body,
    mesh=_mesh,
    out_type=jax.ShapeDtypeStruct((NC, NPAD, D), jnp.float32),
    scratch_types=[
        pltpu.VMEM((CH, LANES), jnp.int32),
        pltpu.VMEM((CH, LANES), jnp.int32),
        [pltpu.VMEM((LANES, D), jnp.float32) for _ in range(NBUF)],
        pltpu.VMEM_SHARED((NPAD, D), jnp.float32),
        [pltpu.SemaphoreType.DMA for _ in range(NBUF)],
        [pltpu.SemaphoreType.DMA for _ in range(NBUF)],
    ],
    compiler_params=pltpu.CompilerParams(use_tc_tiling_on_sc=False),
)


def _deg_body2(edges_hbm, out_hbm, dst_idx, ones_v, zbuf, acc):
    cid = lax.axis_index("c")
    sid = lax.axis_index("s")
    wid = sid * NC + cid

    def fill(i, carry):
        zbuf[pl.ds(i * 16, 16)] = jnp.zeros((16,), jnp.float32)
        return carry

    lax.fori_loop(0, RPS // 16, fill, 0)
    for k in range(LANES // 16):
        ones_v[pl.ds(k * 16, 16)] = jnp.ones((16,), jnp.float32)

    pltpu.sync_copy(zbuf, acc.at[pl.ds(sid * RPS, RPS)])
    plsc.subcore_barrier()

    pltpu.sync_copy(edges_hbm.at[1, wid], dst_idx)

    def step(j, carry):
        pltpu.sync_copy(ones_v, acc.at[dst_idx.at[j]], add=True)
        return carry

    lax.fori_loop(0, CH, step, 0)
    plsc.subcore_barrier()

    pltpu.sync_copy(acc.at[pl.ds(sid * RPS, RPS)],
                    out_hbm.at[cid, pl.ds(sid * RPS, RPS)])


_deg_kernel = pl.kernel(
    _deg_body2,
    mesh=_mesh,
    out_type=jax.ShapeDtypeStruct((NC, NPAD), jnp.float32),
    scratch_types=[
        pltpu.VMEM((CH, LANES), jnp.int32),
        pltpu.VMEM((LANES,), jnp.float32),
        pltpu.VMEM((RPS,), jnp.float32),
        pltpu.VMEM_SHARED((NPAD,), jnp.float32),
    ],
    compiler_params=pltpu.CompilerParams(use_tc_tiling_on_sc=False),
)


def _mm_body(x_ref, w_ref, o_ref):
    o_ref[...] = lax.dot_general(
        x_ref[...], w_ref[...], (((1,), (0,)), ((), ())),
        preferred_element_type=jnp.float32,
        precision=lax.Precision.HIGHEST)


_mm = pl.pallas_call(
    _mm_body,
    grid=(GRID,),
    in_specs=[
        pl.BlockSpec((BLK, DIN), lambda i: (i, 0)),
        pl.BlockSpec((DIN, D), lambda i: (0, 0)),
    ],
    out_specs=pl.BlockSpec((BLK, D), lambda i: (i, 0)),
    out_shape=jax.ShapeDtypeStruct((NPAD, D), jnp.float32),
)


def _scale_body(deg_ref, xw_ref, t_ref, dinv_ref):
    degt = deg_ref[0] + deg_ref[1] + 1.0
    dinv = lax.rsqrt(degt)
    dinv_ref[...] = dinv
    t_ref[...] = xw_ref[...] * dinv


_scale = pl.pallas_call(
    _scale_body,
    grid=(GRID,),
    in_specs=[
        pl.BlockSpec((NC, BLK, 1), lambda i: (0, i, 0)),
        pl.BlockSpec((BLK, D), lambda i: (i, 0)),
    ],
    out_specs=[
        pl.BlockSpec((BLK, D), lambda i: (i, 0)),
        pl.BlockSpec((BLK, 1), lambda i: (i, 0)),
    ],
    out_shape=[
        jax.ShapeDtypeStruct((NPAD, D), jnp.float32),
        jax.ShapeDtypeStruct((NPAD, 1), jnp.float32),
    ],
)


def _layer_body(t1_ref, seg_ref, dinv_ref, b_ref, w_ref, t2_ref):
    dinv = dinv_ref[...]
    seg = seg_ref[0] + seg_ref[1]
    pre = (seg + t1_ref[...]) * dinv + b_ref[...][None, :]
    h = jnp.maximum(pre, 0.0)
    xw2 = lax.dot_general(
        h, w_ref[...], (((1,), (0,)), ((), ())),
        preferred_element_type=jnp.float32,
        precision=lax.Precision.HIGHEST)
    t2_ref[...] = xw2 * dinv


_layer = pl.pallas_call(
    _layer_body,
    grid=(GRID,),
    in_specs=[
        pl.BlockSpec((BLK, D), lambda i: (i, 0)),
        pl.BlockSpec((NC, BLK, D), lambda i: (0, i, 0)),
        pl.BlockSpec((BLK, 1), lambda i: (i, 0)),
        pl.BlockSpec((D,), lambda i: (0,)),
        pl.BlockSpec((D, D), lambda i: (0, 0)),
    ],
    out_specs=pl.BlockSpec((BLK, D), lambda i: (i, 0)),
    out_shape=jax.ShapeDtypeStruct((NPAD, D), jnp.float32),
)


def _final_body(t2_ref, seg_ref, dinv_ref, b_ref, o_ref):
    i = pl.program_id(0)
    seg = seg_ref[0] + seg_ref[1]
    pre = (seg + t2_ref[...]) * dinv_ref[...] + b_ref[...][None, :]
    h = jnp.maximum(pre, 0.0)
    row = lax.broadcasted_iota(jnp.int32, (BLK, 1), 0) + i * BLK
    h = jnp.where(row < N, h, 0.0)
    s = jnp.sum(h, axis=0) * (1.0 / N)

    @pl.when(i == 0)
    def _init():
        o_ref[...] = s

    @pl.when(i > 0)
    def _acc():
        o_ref[...] = o_ref[...] + s


_final = pl.pallas_call(
    _final_body,
    grid=(GRID,),
    in_specs=[
        pl.BlockSpec((BLK, D), lambda i: (i, 0)),
        pl.BlockSpec((NC, BLK, D), lambda i: (0, i, 0)),
        pl.BlockSpec((BLK, 1), lambda i: (i, 0)),
        pl.BlockSpec((D,), lambda i: (0,)),
    ],
    out_specs=pl.BlockSpec((D,), lambda i: (0,)),
    out_shape=jax.ShapeDtypeStruct((D,), jnp.float32),
)


def kernel(x, edge_index, W1, b1, W2, b2):
    pad = (jnp.arange(EPAD - E, dtype=jnp.int32) % (NPAD - N)) + N
    edges = jnp.concatenate(
        [edge_index, jnp.broadcast_to(pad, (2, EPAD - E))], axis=1
    ).reshape(2, NW, CH, LANES)
    xp = jnp.pad(x, ((0, NPAD - N), (0, 0)))

    deg = _deg_kernel(edges)
    xw1 = _mm(xp, W1)
    t1, dinv = _scale(jnp.reshape(deg, (NC, NPAD, 1)), xw1)
    seg1 = _seg_kernel(t1, edges)
    t2 = _layer(t1, seg1, dinv, b1, W2)
    seg2 = _seg_kernel(t2, edges)
    return _final(t2, seg2, dinv, b2)
